# baseline (speedup 1.0000x reference)
import functools

import jax
import jax.numpy as jnp
from jax import lax
from jax.experimental import pallas as pl
from jax.experimental.pallas import tpu as pltpu
from jax.experimental.pallas import tpu_sc as plsc

N = 10000
NP = 10240
E = 320000
D = 128

NC = 2
NS = 16
NW = NC * NS
EPT = E // NW
K = 40
EPTP = 10240
EP = NW * EPTP
NCHUNK = EPTP // K
NBUF = 4
NGROUP = NCHUNK // NBUF

KD = 128
NCHF = EPT // KD
KT = EPT - NCHF * KD
NBUFD = 6
NGROUPD = NCHF // NBUFD

RPT = NP // NS
ZR = 16

BT = 1000
GRID = N // BT

_mesh = plsc.VectorSubcoreMesh(core_axis_name="c", subcore_axis_name="s")
_sc_params = pltpu.CompilerParams(use_tc_tiling_on_sc=False)


@functools.partial(
    pl.kernel,
    out_type=jax.ShapeDtypeStruct((NC, 2, NP), jnp.float32),
    mesh=_mesh,
    scratch_types=[
        pltpu.VMEM((KD,), jnp.int32),
        pltpu.VMEM((KD,), jnp.int32),
        pltpu.VMEM((KD,), jnp.int32),
        pltpu.VMEM((KD,), jnp.int32),
        pltpu.VMEM((KD,), jnp.int32),
        pltpu.VMEM((KD,), jnp.int32),
        pltpu.VMEM((KD,), jnp.int32),
        pltpu.VMEM((KD,), jnp.int32),
        pltpu.VMEM((KD,), jnp.int32),
        pltpu.VMEM((KD,), jnp.int32),
        pltpu.VMEM((KD,), jnp.int32),
        pltpu.VMEM((KD,), jnp.int32),
        pltpu.VMEM((KT,), jnp.int32),
        pltpu.VMEM((KT,), jnp.int32),
        pltpu.VMEM((KD,), jnp.float32),
        pltpu.VMEM((RPT,), jnp.float32),
        pltpu.VMEM_SHARED((NP,), jnp.float32),
        pltpu.VMEM_SHARED((NP,), jnp.float32),
        pltpu.SemaphoreType.DMA,
        pltpu.SemaphoreType.DMA,
        pltpu.SemaphoreType.DMA,
        pltpu.SemaphoreType.DMA,
        pltpu.SemaphoreType.DMA,
        pltpu.SemaphoreType.DMA,
    ],
)
def _degree_kernel(src_hbm, dst_hbm, out_hbm, s0, s1, s2, s3, s4, s5,
                   e0, e1, e2, e3, e4, e5, stail, dtail,
                   ones, zbuf, dsrc, ddst, l0, l1, l2, l3, l4, l5):
    c = lax.axis_index("c")
    s = lax.axis_index("s")
    w = s * NC + c
    sbuf = (s0, s1, s2, s3, s4, s5)
    dbuf = (e0, e1, e2, e3, e4, e5)
    lsem = (l0, l1, l2, l3, l4, l5)
    base = w * EPT

    for b in range(NBUFD):
        pltpu.async_copy(src_hbm.at[pl.ds(base + b * KD, KD)],
                         sbuf[b], lsem[b])
        pltpu.async_copy(dst_hbm.at[pl.ds(base + b * KD, KD)],
                         dbuf[b], lsem[b])

    def fill(i, _):
        ones[pl.ds(i * 16, 16)] = jnp.ones((16,), jnp.float32)
        return 0

    lax.fori_loop(0, KD // 16, fill, 0)

    def zfill(i, _):
        zbuf[pl.ds(i * 16, 16)] = jnp.zeros((16,), jnp.float32)
        return 0

    lax.fori_loop(0, RPT // 16, zfill, 0)

    row0 = s * RPT
    pltpu.sync_copy(zbuf, dsrc.at[pl.ds(row0, RPT)])
    pltpu.sync_copy(zbuf, ddst.at[pl.ds(row0, RPT)])
    plsc.subcore_barrier()

    def body(g, _):
        for b in range(NBUFD):
            j = g * NBUFD + b
            pltpu.make_async_copy(src_hbm.at[pl.ds(0, KD)], sbuf[b],
                                  lsem[b]).wait()
            pltpu.make_async_copy(dst_hbm.at[pl.ds(0, KD)], dbuf[b],
                                  lsem[b]).wait()
            pltpu.sync_copy(ones, dsrc.at[sbuf[b]], add=True)
            pltpu.sync_copy(ones, ddst.at[dbuf[b]], add=True)

            @pl.when(g < NGROUPD - 1)
            def _():
                jn = j + NBUFD
                pltpu.async_copy(src_hbm.at[pl.ds(base + jn * KD, KD)],
                                 sbuf[b], lsem[b])
                pltpu.async_copy(dst_hbm.at[pl.ds(base + jn * KD, KD)],
                                 dbuf[b], lsem[b])

        return 0

    lax.fori_loop(0, NGROUPD, body, 0)

    toff = base + NCHF * KD
    pltpu.sync_copy(src_hbm.at[pl.ds(toff, KT)], stail)
    pltpu.sync_copy(dst_hbm.at[pl.ds(toff, KT)], dtail)
    pltpu.sync_copy(ones.at[pl.ds(0, KT)], dsrc.at[stail], add=True)
    pltpu.sync_copy(ones.at[pl.ds(0, KT)], ddst.at[dtail], add=True)
    plsc.subcore_barrier()

    pltpu.sync_copy(dsrc.at[pl.ds(row0, RPT)], out_hbm.at[c, 0, pl.ds(row0, RPT)])
    pltpu.sync_copy(ddst.at[pl.ds(row0, RPT)], out_hbm.at[c, 1, pl.ds(row0, RPT)])


@functools.partial(
    pl.kernel,
    out_type=jax.ShapeDtypeStruct((NC, NP, D), jnp.float32),
    mesh=_mesh,
    compiler_params=_sc_params,
    scratch_types=[
        pltpu.VMEM((EPTP,), jnp.int32),
        pltpu.VMEM((K,), jnp.int32),
        pltpu.VMEM((K,), jnp.int32),
        pltpu.VMEM((K,), jnp.int32),
        pltpu.VMEM((K,), jnp.int32),
        pltpu.VMEM((K, D), jnp.float32),
        pltpu.VMEM((K, D), jnp.float32),
        pltpu.VMEM((K, D), jnp.float32),
        pltpu.VMEM((K, D), jnp.float32),
        pltpu.VMEM((ZR, D), jnp.float32),
        pltpu.VMEM_SHARED((NP, D), jnp.float32),
        pltpu.SemaphoreType.DMA,
        pltpu.SemaphoreType.DMA,
        pltpu.SemaphoreType.DMA,
        pltpu.SemaphoreType.DMA,
        pltpu.SemaphoreType.DMA,
        pltpu.SemaphoreType.DMA,
        pltpu.SemaphoreType.DMA,
        pltpu.SemaphoreType.DMA,
        pltpu.SemaphoreType.DMA,
    ],
)
def _edge_kernel(h_hbm, src_hbm, dst_hbm, out_hbm, sidx,
                 d0, d1, d2, d3, r0, r1, r2, r3, zbuf, agg,
                 i0, i1, i2, i3, g0, g1, g2, g3, psem):
    c = lax.axis_index("c")
    s = lax.axis_index("s")
    w = s * NC + c
    dbuf = (d0, d1, d2, d3)
    rows = (r0, r1, r2, r3)
    isem = (i0, i1, i2, i3)
    gsem = (g0, g1, g2, g3)
    base = w * EPTP

    pltpu.async_copy(src_hbm.at[pl.ds(base, EPTP)], sidx, psem)

    def zrow(i, _):
        def zcol(q, _):
            zbuf[i, pl.ds(q * 16, 16)] = jnp.zeros((16,), jnp.float32)
            return 0

        lax.fori_loop(0, D // 16, zcol, 0)
        return 0

    lax.fori_loop(0, ZR, zrow, 0)

    row0 = s * RPT
    for j in range(RPT // ZR):
        pltpu.sync_copy(zbuf, agg.at[pl.ds(row0 + j * ZR, ZR)])

    pltpu.make_async_copy(src_hbm.at[pl.ds(base, EPTP)], sidx, psem).wait()

    for b in range(NBUF):
        pltpu.async_copy(dst_hbm.at[pl.ds(base + b * K, K)], dbuf[b], isem[b])
        pltpu.async_copy(h_hbm.at[sidx.at[pl.ds(b * K, K)]], rows[b], gsem[b])
    plsc.subcore_barrier()

    def body(g, _):
        for b in range(NBUF):
            j = g * NBUF + b
            pltpu.make_async_copy(h_hbm.at[pl.ds(0, K)], rows[b], gsem[b]).wait()
            pltpu.make_async_copy(dst_hbm.at[pl.ds(0, K)], dbuf[b], isem[b]).wait()
            pltpu.sync_copy(rows[b], agg.at[dbuf[b]], add=True)

            @pl.when(g < NGROUP - 1)
            def _():
                jn = j + NBUF
                pltpu.async_copy(dst_hbm.at[pl.ds(base + jn * K, K)],
                                 dbuf[b], isem[b])
                pltpu.async_copy(h_hbm.at[sidx.at[pl.ds(jn * K, K)]],
                                 rows[b], gsem[b])

        return 0

    lax.fori_loop(0, NGROUP, body, 0)
    plsc.subcore_barrier()

    pltpu.sync_copy(agg.at[pl.ds(row0, RPT)], out_hbm.at[c, pl.ds(row0, RPT)])


def _norms(dg):
    dsrc = dg[0, 0] + dg[1, 0]
    ddst = dg[0, 1] + dg[1, 1]
    nsrc = lax.rsqrt(jnp.maximum(dsrc, 1.0))
    ndst = lax.rsqrt(jnp.maximum(ddst, 1.0))
    return nsrc, ndst


def _tc0_body(x_ref, dg_ref, w_ref, o_ref):
    nsrc, _ = _norms(dg_ref[...])
    o_ref[...] = jnp.dot(
        x_ref[...] * nsrc, w_ref[...], preferred_element_type=jnp.float32
    )


def _tcmid_body(ap_ref, dg_ref, b_ref, w_ref, o_ref):
    nsrc, ndst = _norms(dg_ref[...])
    agg = ap_ref[0] + ap_ref[1]
    z = jnp.maximum(agg * ndst + b_ref[...], 0.0)
    o_ref[...] = jnp.dot(z * nsrc, w_ref[...], preferred_element_type=jnp.float32)


def _tcfin_body(ap_ref, dg_ref, b_ref, o_ref):
    _, ndst = _norms(dg_ref[...])
    o_ref[...] = (ap_ref[0] + ap_ref[1]) * ndst + b_ref[...]


_dg_spec = pl.BlockSpec((2, 2, BT, 1), lambda j: (0, 0, j, 0))
_row_spec = pl.BlockSpec((BT, D), lambda j: (j, 0))
_ap_spec = pl.BlockSpec((2, BT, D), lambda j: (0, j, 0))
_w_spec = pl.BlockSpec((D, D), lambda j: (0, 0))
_b_spec = pl.BlockSpec((1, D), lambda j: (0, 0))

_tc0 = pl.pallas_call(
    _tc0_body,
    grid=(GRID,),
    in_specs=[_row_spec, _dg_spec, _w_spec],
    out_specs=_row_spec,
    out_shape=jax.ShapeDtypeStruct((N, D), jnp.float32),
)

_tcmid = pl.pallas_call(
    _tcmid_body,
    grid=(GRID,),
    in_specs=[_ap_spec, _dg_spec, _b_spec, _w_spec],
    out_specs=_row_spec,
    out_shape=jax.ShapeDtypeStruct((N, D), jnp.float32),
)

_tcfin = pl.pallas_call(
    _tcfin_body,
    grid=(GRID,),
    in_specs=[_ap_spec, _dg_spec, _b_spec],
    out_specs=_row_spec,
    out_shape=jax.ShapeDtypeStruct((N, D), jnp.float32),
)


@jax.jit
def kernel(x, edge_index, W0, b0, W1, b1):
    src = edge_index[0]
    dst = edge_index[1]

    ppt = EPTP - EPT
    pad_dst = jnp.broadcast_to(N + jnp.arange(ppt, dtype=jnp.int32), (NW, ppt))
    src_p = jnp.concatenate(
        [src.reshape(NW, EPT), jnp.zeros((NW, ppt), jnp.int32)], axis=1
    ).reshape(EP)
    dst_p = jnp.concatenate([dst.reshape(NW, EPT), pad_dst], axis=1).reshape(EP)

    deg = _degree_kernel(src, dst)
    dg = deg[:, :, :N].reshape(NC, 2, N, 1)

    h0 = _tc0(x, dg, W0)
    agg0 = _edge_kernel(h0, src_p, dst_p)
    h1 = _tcmid(agg0, dg, b0.reshape(1, D), W1)
    agg1 = _edge_kernel(h1, src_p, dst_p)
    return _tcfin(agg1, dg, b1.reshape(1, D))


# speedup vs baseline: 3.0741x; 3.0741x over previous
import functools

import jax
import jax.numpy as jnp
from jax import lax
from jax.experimental import pallas as pl
from jax.experimental.pallas import tpu as pltpu
from jax.experimental.pallas import tpu_sc as plsc

N = 10000
NP = 10240
E = 320000
D = 128

NC = 2
NS = 16
NW = NC * NS
EPT = E // NW
K = 40
NCHUNK = EPT // K
NBUF = 5
NGROUP = NCHUNK // NBUF

KD = 128
NCHF = EPT // KD
KT = EPT - NCHF * KD
NBUFD = 6
NGROUPD = NCHF // NBUFD

RPT = NP // NS
ZR = 16

BT = 1000
GRID = N // BT

_mesh = plsc.VectorSubcoreMesh(core_axis_name="c", subcore_axis_name="s")
_sc_params = pltpu.CompilerParams(use_tc_tiling_on_sc=False)


@functools.partial(
    pl.kernel,
    out_type=jax.ShapeDtypeStruct((NC, 2, NP), jnp.float32),
    mesh=_mesh,
    scratch_types=[
        pltpu.VMEM((KD,), jnp.int32),
        pltpu.VMEM((KD,), jnp.int32),
        pltpu.VMEM((KD,), jnp.int32),
        pltpu.VMEM((KD,), jnp.int32),
        pltpu.VMEM((KD,), jnp.int32),
        pltpu.VMEM((KD,), jnp.int32),
        pltpu.VMEM((KD,), jnp.int32),
        pltpu.VMEM((KD,), jnp.int32),
        pltpu.VMEM((KD,), jnp.int32),
        pltpu.VMEM((KD,), jnp.int32),
        pltpu.VMEM((KD,), jnp.int32),
        pltpu.VMEM((KD,), jnp.int32),
        pltpu.VMEM((KT,), jnp.int32),
        pltpu.VMEM((KT,), jnp.int32),
        pltpu.VMEM((KD,), jnp.float32),
        pltpu.VMEM((RPT,), jnp.float32),
        pltpu.VMEM_SHARED((NP,), jnp.float32),
        pltpu.VMEM_SHARED((NP,), jnp.float32),
        pltpu.SemaphoreType.DMA,
        pltpu.SemaphoreType.DMA,
        pltpu.SemaphoreType.DMA,
        pltpu.SemaphoreType.DMA,
        pltpu.SemaphoreType.DMA,
        pltpu.SemaphoreType.DMA,
    ],
)
def _degree_kernel(src_hbm, dst_hbm, out_hbm, s0, s1, s2, s3, s4, s5,
                   e0, e1, e2, e3, e4, e5, stail, dtail,
                   ones, zbuf, dsrc, ddst, l0, l1, l2, l3, l4, l5):
    c = lax.axis_index("c")
    s = lax.axis_index("s")
    w = s * NC + c
    sbuf = (s0, s1, s2, s3, s4, s5)
    dbuf = (e0, e1, e2, e3, e4, e5)
    lsem = (l0, l1, l2, l3, l4, l5)
    base = w * EPT

    for b in range(NBUFD):
        pltpu.async_copy(src_hbm.at[pl.ds(base + b * KD, KD)],
                         sbuf[b], lsem[b])
        pltpu.async_copy(dst_hbm.at[pl.ds(base + b * KD, KD)],
                         dbuf[b], lsem[b])

    def fill(i, _):
        ones[pl.ds(i * 16, 16)] = jnp.ones((16,), jnp.float32)
        return 0

    lax.fori_loop(0, KD // 16, fill, 0)

    def zfill(i, _):
        zbuf[pl.ds(i * 16, 16)] = jnp.zeros((16,), jnp.float32)
        return 0

    lax.fori_loop(0, RPT // 16, zfill, 0)

    row0 = s * RPT
    pltpu.sync_copy(zbuf, dsrc.at[pl.ds(row0, RPT)])
    pltpu.sync_copy(zbuf, ddst.at[pl.ds(row0, RPT)])
    plsc.subcore_barrier()

    def body(g, _):
        for b in range(NBUFD):
            j = g * NBUFD + b
            pltpu.make_async_copy(src_hbm.at[pl.ds(0, KD)], sbuf[b],
                                  lsem[b]).wait()
            pltpu.make_async_copy(dst_hbm.at[pl.ds(0, KD)], dbuf[b],
                                  lsem[b]).wait()
            pltpu.sync_copy(ones, dsrc.at[sbuf[b]], add=True)
            pltpu.sync_copy(ones, ddst.at[dbuf[b]], add=True)

            @pl.when(g < NGROUPD - 1)
            def _():
                jn = j + NBUFD
                pltpu.async_copy(src_hbm.at[pl.ds(base + jn * KD, KD)],
                                 sbuf[b], lsem[b])
                pltpu.async_copy(dst_hbm.at[pl.ds(base + jn * KD, KD)],
                                 dbuf[b], lsem[b])

        return 0

    lax.fori_loop(0, NGROUPD, body, 0)

    toff = base + NCHF * KD
    pltpu.sync_copy(src_hbm.at[pl.ds(toff, KT)], stail)
    pltpu.sync_copy(dst_hbm.at[pl.ds(toff, KT)], dtail)
    pltpu.sync_copy(ones.at[pl.ds(0, KT)], dsrc.at[stail], add=True)
    pltpu.sync_copy(ones.at[pl.ds(0, KT)], ddst.at[dtail], add=True)
    plsc.subcore_barrier()

    pltpu.sync_copy(dsrc.at[pl.ds(row0, RPT)], out_hbm.at[c, 0, pl.ds(row0, RPT)])
    pltpu.sync_copy(ddst.at[pl.ds(row0, RPT)], out_hbm.at[c, 1, pl.ds(row0, RPT)])


@functools.partial(
    pl.kernel,
    out_type=jax.ShapeDtypeStruct((NC, NP, D), jnp.float32),
    mesh=_mesh,
    compiler_params=_sc_params,
    scratch_types=[
        pltpu.VMEM((EPT,), jnp.int32),
        pltpu.VMEM((K,), jnp.int32),
        pltpu.VMEM((K,), jnp.int32),
        pltpu.VMEM((K,), jnp.int32),
        pltpu.VMEM((K,), jnp.int32),
        pltpu.VMEM((K,), jnp.int32),
        pltpu.VMEM((K, D), jnp.float32),
        pltpu.VMEM((K, D), jnp.float32),
        pltpu.VMEM((K, D), jnp.float32),
        pltpu.VMEM((K, D), jnp.float32),
        pltpu.VMEM((K, D), jnp.float32),
        pltpu.VMEM((ZR, D), jnp.float32),
        pltpu.VMEM_SHARED((NP, D), jnp.float32),
        pltpu.SemaphoreType.DMA,
        pltpu.SemaphoreType.DMA,
        pltpu.SemaphoreType.DMA,
        pltpu.SemaphoreType.DMA,
        pltpu.SemaphoreType.DMA,
        pltpu.SemaphoreType.DMA,
        pltpu.SemaphoreType.DMA,
        pltpu.SemaphoreType.DMA,
        pltpu.SemaphoreType.DMA,
        pltpu.SemaphoreType.DMA,
        pltpu.SemaphoreType.DMA,
    ],
)
def _edge_kernel(h_hbm, src_hbm, dst_hbm, out_hbm, sidx,
                 d0, d1, d2, d3, d4, r0, r1, r2, r3, r4, zbuf, agg,
                 i0, i1, i2, i3, i4, g0, g1, g2, g3, g4, psem):
    c = lax.axis_index("c")
    s = lax.axis_index("s")
    w = s * NC + c
    dbuf = (d0, d1, d2, d3, d4)
    rows = (r0, r1, r2, r3, r4)
    isem = (i0, i1, i2, i3, i4)
    gsem = (g0, g1, g2, g3, g4)
    base = w * EPT

    pltpu.async_copy(src_hbm.at[pl.ds(base, EPT)], sidx, psem)

    def zrow(i, _):
        def zcol(q, _):
            zbuf[i, pl.ds(q * 16, 16)] = jnp.zeros((16,), jnp.float32)
            return 0

        lax.fori_loop(0, D // 16, zcol, 0)
        return 0

    lax.fori_loop(0, ZR, zrow, 0)

    row0 = s * RPT
    for j in range(RPT // ZR):
        pltpu.sync_copy(zbuf, agg.at[pl.ds(row0 + j * ZR, ZR)])

    pltpu.make_async_copy(src_hbm.at[pl.ds(base, EPT)], sidx, psem).wait()

    for b in range(NBUF):
        pltpu.async_copy(dst_hbm.at[pl.ds(base + b * K, K)], dbuf[b], isem[b])
        pltpu.async_copy(h_hbm.at[sidx.at[pl.ds(b * K, K)]], rows[b], gsem[b])
    plsc.subcore_barrier()

    def body(g, _):
        for b in range(NBUF):
            j = g * NBUF + b
            pltpu.make_async_copy(h_hbm.at[pl.ds(0, K)], rows[b], gsem[b]).wait()
            pltpu.make_async_copy(dst_hbm.at[pl.ds(0, K)], dbuf[b], isem[b]).wait()
            pltpu.sync_copy(rows[b], agg.at[dbuf[b]], add=True)

            @pl.when(g < NGROUP - 1)
            def _():
                jn = j + NBUF
                pltpu.async_copy(dst_hbm.at[pl.ds(base + jn * K, K)],
                                 dbuf[b], isem[b])
                pltpu.async_copy(h_hbm.at[sidx.at[pl.ds(jn * K, K)]],
                                 rows[b], gsem[b])

        return 0

    lax.fori_loop(0, NGROUP, body, 0)
    plsc.subcore_barrier()

    pltpu.sync_copy(agg.at[pl.ds(row0, RPT)], out_hbm.at[c, pl.ds(row0, RPT)])


def _norms(dg):
    dsrc = dg[0, 0] + dg[1, 0]
    ddst = dg[0, 1] + dg[1, 1]
    nsrc = lax.rsqrt(jnp.maximum(dsrc, 1.0))
    ndst = lax.rsqrt(jnp.maximum(ddst, 1.0))
    return nsrc, ndst


def _tc0_body(x_ref, dg_ref, w_ref, o_ref):
    nsrc, _ = _norms(dg_ref[...])
    o_ref[...] = jnp.dot(
        x_ref[...] * nsrc, w_ref[...], preferred_element_type=jnp.float32
    )


def _tcmid_body(ap_ref, dg_ref, b_ref, w_ref, o_ref):
    nsrc, ndst = _norms(dg_ref[...])
    agg = ap_ref[0] + ap_ref[1]
    z = jnp.maximum(agg * ndst + b_ref[...], 0.0)
    o_ref[...] = jnp.dot(z * nsrc, w_ref[...], preferred_element_type=jnp.float32)


def _tcfin_body(ap_ref, dg_ref, b_ref, o_ref):
    _, ndst = _norms(dg_ref[...])
    o_ref[...] = (ap_ref[0] + ap_ref[1]) * ndst + b_ref[...]


_dg_spec = pl.BlockSpec((2, 2, BT, 1), lambda j: (0, 0, j, 0))
_row_spec = pl.BlockSpec((BT, D), lambda j: (j, 0))
_ap_spec = pl.BlockSpec((2, BT, D), lambda j: (0, j, 0))
_w_spec = pl.BlockSpec((D, D), lambda j: (0, 0))
_b_spec = pl.BlockSpec((1, D), lambda j: (0, 0))

_tc0 = pl.pallas_call(
    _tc0_body,
    grid=(GRID,),
    in_specs=[_row_spec, _dg_spec, _w_spec],
    out_specs=_row_spec,
    out_shape=jax.ShapeDtypeStruct((N, D), jnp.float32),
)

_tcmid = pl.pallas_call(
    _tcmid_body,
    grid=(GRID,),
    in_specs=[_ap_spec, _dg_spec, _b_spec, _w_spec],
    out_specs=_row_spec,
    out_shape=jax.ShapeDtypeStruct((N, D), jnp.float32),
)

_tcfin = pl.pallas_call(
    _tcfin_body,
    grid=(GRID,),
    in_specs=[_ap_spec, _dg_spec, _b_spec],
    out_specs=_row_spec,
    out_shape=jax.ShapeDtypeStruct((N, D), jnp.float32),
)


@jax.jit
def kernel(x, edge_index, W0, b0, W1, b1):
    src = edge_index[0]
    dst = edge_index[1]

    deg = _degree_kernel(src, dst)
    dg = deg[:, :, :N].reshape(NC, 2, N, 1)

    h0 = _tc0(x, dg, W0)
    agg0 = _edge_kernel(h0, src, dst)
    h1 = _tcmid(agg0, dg, b0.reshape(1, D), W1)
    agg1 = _edge_kernel(h1, src, dst)
    return _tcfin(agg1, dg, b1.reshape(1, D))


# speedup vs baseline: 3.1335x; 1.0193x over previous
import functools

import jax
import jax.numpy as jnp
from jax import lax
from jax.experimental import pallas as pl
from jax.experimental.pallas import tpu as pltpu
from jax.experimental.pallas import tpu_sc as plsc

N = 10000
NP = 10240
E = 320000
D = 128

NC = 2
NS = 16
NW = NC * NS
EPT = E // NW
K = 40
NCHUNK = EPT // K
NBUF = 5
NGROUP = NCHUNK // NBUF

KD = 128
NCHF = EPT // KD
KT = EPT - NCHF * KD
NBUFD = 6
NGROUPD = NCHF // NBUFD

RPT = NP // NS
ZR = 16

BT = 2000
GRID = N // BT

_mesh = plsc.VectorSubcoreMesh(core_axis_name="c", subcore_axis_name="s")
_sc_params = pltpu.CompilerParams(use_tc_tiling_on_sc=False)


@functools.partial(
    pl.kernel,
    out_type=jax.ShapeDtypeStruct((NC, 2, NP), jnp.float32),
    mesh=_mesh,
    scratch_types=[
        pltpu.VMEM((KD,), jnp.int32),
        pltpu.VMEM((KD,), jnp.int32),
        pltpu.VMEM((KD,), jnp.int32),
        pltpu.VMEM((KD,), jnp.int32),
        pltpu.VMEM((KD,), jnp.int32),
        pltpu.VMEM((KD,), jnp.int32),
        pltpu.VMEM((KD,), jnp.int32),
        pltpu.VMEM((KD,), jnp.int32),
        pltpu.VMEM((KD,), jnp.int32),
        pltpu.VMEM((KD,), jnp.int32),
        pltpu.VMEM((KD,), jnp.int32),
        pltpu.VMEM((KD,), jnp.int32),
        pltpu.VMEM((KT,), jnp.int32),
        pltpu.VMEM((KT,), jnp.int32),
        pltpu.VMEM((KD,), jnp.float32),
        pltpu.VMEM((RPT,), jnp.float32),
        pltpu.VMEM_SHARED((NP,), jnp.float32),
        pltpu.VMEM_SHARED((NP,), jnp.float32),
        pltpu.SemaphoreType.DMA,
        pltpu.SemaphoreType.DMA,
        pltpu.SemaphoreType.DMA,
        pltpu.SemaphoreType.DMA,
        pltpu.SemaphoreType.DMA,
        pltpu.SemaphoreType.DMA,
    ],
)
def _degree_kernel(src_hbm, dst_hbm, out_hbm, s0, s1, s2, s3, s4, s5,
                   e0, e1, e2, e3, e4, e5, stail, dtail,
                   ones, zbuf, dsrc, ddst, l0, l1, l2, l3, l4, l5):
    c = lax.axis_index("c")
    s = lax.axis_index("s")
    w = s * NC + c
    sbuf = (s0, s1, s2, s3, s4, s5)
    dbuf = (e0, e1, e2, e3, e4, e5)
    lsem = (l0, l1, l2, l3, l4, l5)
    base = w * EPT

    for b in range(NBUFD):
        pltpu.async_copy(src_hbm.at[pl.ds(base + b * KD, KD)],
                         sbuf[b], lsem[b])
        pltpu.async_copy(dst_hbm.at[pl.ds(base + b * KD, KD)],
                         dbuf[b], lsem[b])

    def fill(i, _):
        ones[pl.ds(i * 16, 16)] = jnp.ones((16,), jnp.float32)
        return 0

    lax.fori_loop(0, KD // 16, fill, 0)

    def zfill(i, _):
        zbuf[pl.ds(i * 16, 16)] = jnp.zeros((16,), jnp.float32)
        return 0

    lax.fori_loop(0, RPT // 16, zfill, 0)

    row0 = s * RPT
    pltpu.sync_copy(zbuf, dsrc.at[pl.ds(row0, RPT)])
    pltpu.sync_copy(zbuf, ddst.at[pl.ds(row0, RPT)])
    plsc.subcore_barrier()

    def body(g, _):
        for b in range(NBUFD):
            j = g * NBUFD + b
            pltpu.make_async_copy(src_hbm.at[pl.ds(0, KD)], sbuf[b],
                                  lsem[b]).wait()
            pltpu.make_async_copy(dst_hbm.at[pl.ds(0, KD)], dbuf[b],
                                  lsem[b]).wait()
            pltpu.sync_copy(ones, dsrc.at[sbuf[b]], add=True)
            pltpu.sync_copy(ones, ddst.at[dbuf[b]], add=True)

            @pl.when(g < NGROUPD - 1)
            def _():
                jn = j + NBUFD
                pltpu.async_copy(src_hbm.at[pl.ds(base + jn * KD, KD)],
                                 sbuf[b], lsem[b])
                pltpu.async_copy(dst_hbm.at[pl.ds(base + jn * KD, KD)],
                                 dbuf[b], lsem[b])

        return 0

    lax.fori_loop(0, NGROUPD, body, 0)

    toff = base + NCHF * KD
    pltpu.sync_copy(src_hbm.at[pl.ds(toff, KT)], stail)
    pltpu.sync_copy(dst_hbm.at[pl.ds(toff, KT)], dtail)
    pltpu.sync_copy(ones.at[pl.ds(0, KT)], dsrc.at[stail], add=True)
    pltpu.sync_copy(ones.at[pl.ds(0, KT)], ddst.at[dtail], add=True)
    plsc.subcore_barrier()

    pltpu.sync_copy(dsrc.at[pl.ds(row0, RPT)], out_hbm.at[c, 0, pl.ds(row0, RPT)])
    pltpu.sync_copy(ddst.at[pl.ds(row0, RPT)], out_hbm.at[c, 1, pl.ds(row0, RPT)])


@functools.partial(
    pl.kernel,
    out_type=jax.ShapeDtypeStruct((NC, NP, D), jnp.float32),
    mesh=_mesh,
    compiler_params=_sc_params,
    scratch_types=[
        pltpu.VMEM((EPT,), jnp.int32),
        pltpu.VMEM((K,), jnp.int32),
        pltpu.VMEM((K,), jnp.int32),
        pltpu.VMEM((K,), jnp.int32),
        pltpu.VMEM((K,), jnp.int32),
        pltpu.VMEM((K,), jnp.int32),
        pltpu.VMEM((K, D), jnp.float32),
        pltpu.VMEM((K, D), jnp.float32),
        pltpu.VMEM((K, D), jnp.float32),
        pltpu.VMEM((K, D), jnp.float32),
        pltpu.VMEM((K, D), jnp.float32),
        pltpu.VMEM((ZR, D), jnp.float32),
        pltpu.VMEM_SHARED((NP, D), jnp.float32),
        pltpu.SemaphoreType.DMA,
        pltpu.SemaphoreType.DMA,
        pltpu.SemaphoreType.DMA,
        pltpu.SemaphoreType.DMA,
        pltpu.SemaphoreType.DMA,
        pltpu.SemaphoreType.DMA,
        pltpu.SemaphoreType.DMA,
        pltpu.SemaphoreType.DMA,
        pltpu.SemaphoreType.DMA,
        pltpu.SemaphoreType.DMA,
        pltpu.SemaphoreType.DMA,
    ],
)
def _edge_kernel(h_hbm, src_hbm, dst_hbm, out_hbm, sidx,
                 d0, d1, d2, d3, d4, r0, r1, r2, r3, r4, zbuf, agg,
                 i0, i1, i2, i3, i4, g0, g1, g2, g3, g4, psem):
    c = lax.axis_index("c")
    s = lax.axis_index("s")
    w = s * NC + c
    dbuf = (d0, d1, d2, d3, d4)
    rows = (r0, r1, r2, r3, r4)
    isem = (i0, i1, i2, i3, i4)
    gsem = (g0, g1, g2, g3, g4)
    base = w * EPT

    pltpu.async_copy(src_hbm.at[pl.ds(base, EPT)], sidx, psem)

    def zrow(i, _):
        def zcol(q, _):
            zbuf[i, pl.ds(q * 16, 16)] = jnp.zeros((16,), jnp.float32)
            return 0

        lax.fori_loop(0, D // 16, zcol, 0)
        return 0

    lax.fori_loop(0, ZR, zrow, 0)

    row0 = s * RPT
    for j in range(RPT // ZR):
        pltpu.sync_copy(zbuf, agg.at[pl.ds(row0 + j * ZR, ZR)])

    pltpu.make_async_copy(src_hbm.at[pl.ds(base, EPT)], sidx, psem).wait()

    for b in range(NBUF):
        pltpu.async_copy(dst_hbm.at[pl.ds(base + b * K, K)], dbuf[b], isem[b])
        pltpu.async_copy(h_hbm.at[sidx.at[pl.ds(b * K, K)]], rows[b], gsem[b])
    plsc.subcore_barrier()

    def body(g, _):
        for b in range(NBUF):
            j = g * NBUF + b
            pltpu.make_async_copy(h_hbm.at[pl.ds(0, K)], rows[b], gsem[b]).wait()
            pltpu.make_async_copy(dst_hbm.at[pl.ds(0, K)], dbuf[b], isem[b]).wait()
            pltpu.sync_copy(rows[b], agg.at[dbuf[b]], add=True)

            @pl.when(g < NGROUP - 1)
            def _():
                jn = j + NBUF
                pltpu.async_copy(dst_hbm.at[pl.ds(base + jn * K, K)],
                                 dbuf[b], isem[b])
                pltpu.async_copy(h_hbm.at[sidx.at[pl.ds(jn * K, K)]],
                                 rows[b], gsem[b])

        return 0

    lax.fori_loop(0, NGROUP, body, 0)
    plsc.subcore_barrier()

    pltpu.sync_copy(agg.at[pl.ds(row0, RPT)], out_hbm.at[c, pl.ds(row0, RPT)])


def _norms(dg):
    dsrc = dg[0, 0] + dg[1, 0]
    ddst = dg[0, 1] + dg[1, 1]
    nsrc = lax.rsqrt(jnp.maximum(dsrc, 1.0))
    ndst = lax.rsqrt(jnp.maximum(ddst, 1.0))
    return nsrc, ndst


def _tc0_body(x_ref, dg_ref, w_ref, o_ref):
    nsrc, _ = _norms(dg_ref[...])
    o_ref[...] = jnp.dot(
        x_ref[...] * nsrc, w_ref[...], preferred_element_type=jnp.float32
    )


def _tcmid_body(ap_ref, dg_ref, b_ref, w_ref, o_ref):
    nsrc, ndst = _norms(dg_ref[...])
    agg = ap_ref[0] + ap_ref[1]
    z = jnp.maximum(agg * ndst + b_ref[...], 0.0)
    o_ref[...] = jnp.dot(z * nsrc, w_ref[...], preferred_element_type=jnp.float32)


def _tcfin_body(ap_ref, dg_ref, b_ref, o_ref):
    _, ndst = _norms(dg_ref[...])
    o_ref[...] = (ap_ref[0] + ap_ref[1]) * ndst + b_ref[...]


_dg_spec = pl.BlockSpec((2, 2, BT, 1), lambda j: (0, 0, j, 0))
_row_spec = pl.BlockSpec((BT, D), lambda j: (j, 0))
_ap_spec = pl.BlockSpec((2, BT, D), lambda j: (0, j, 0))
_w_spec = pl.BlockSpec((D, D), lambda j: (0, 0))
_b_spec = pl.BlockSpec((1, D), lambda j: (0, 0))

_tc0 = pl.pallas_call(
    _tc0_body,
    grid=(GRID,),
    in_specs=[_row_spec, _dg_spec, _w_spec],
    out_specs=_row_spec,
    out_shape=jax.ShapeDtypeStruct((N, D), jnp.float32),
)

_tcmid = pl.pallas_call(
    _tcmid_body,
    grid=(GRID,),
    in_specs=[_ap_spec, _dg_spec, _b_spec, _w_spec],
    out_specs=_row_spec,
    out_shape=jax.ShapeDtypeStruct((N, D), jnp.float32),
)

_tcfin = pl.pallas_call(
    _tcfin_body,
    grid=(GRID,),
    in_specs=[_ap_spec, _dg_spec, _b_spec],
    out_specs=_row_spec,
    out_shape=jax.ShapeDtypeStruct((N, D), jnp.float32),
)


@jax.jit
def kernel(x, edge_index, W0, b0, W1, b1):
    src = edge_index[0]
    dst = edge_index[1]

    deg = _degree_kernel(src, dst)
    dg = deg[:, :, :N].reshape(NC, 2, N, 1)

    h0 = _tc0(x, dg, W0)
    agg0 = _edge_kernel(h0, src, dst)
    h1 = _tcmid(agg0, dg, b0.reshape(1, D), W1)
    agg1 = _edge_kernel(h1, src, dst)
    return _tcfin(agg1, dg, b1.reshape(1, D))
